# baseline (speedup 1.0000x reference)
import functools

import jax
import jax.numpy as jnp
from jax import lax
from jax.experimental import pallas as pl
from jax.experimental.pallas import tpu as pltpu
from jax.experimental.pallas import tpu_sc as plsc

_NC = 2
_NS = 16
_NW = _NC * _NS
_KH = 64


def _sc_body(x0_hbm, noise_hbm, ts_hbm, sac_hbm, somac_hbm, out_hbm,
             idx_v, tval_v, sac_v, somac_v, bufx, bufn, bufo):
    B, C, H, W = x0_hbm.shape
    per_w = B // _NW
    wid = lax.axis_index("s") * _NC + lax.axis_index("c")

    for k in range(per_w):
        b = wid * per_w + k
        idx_v[...] = jnp.full((16,), b, dtype=jnp.int32)
        pltpu.sync_copy(ts_hbm.at[idx_v], tval_v)
        pltpu.sync_copy(sac_hbm.at[tval_v], sac_v)
        pltpu.sync_copy(somac_hbm.at[tval_v], somac_v)
        sac = sac_v[...]
        somac = somac_v[...]

        for c in range(C):
            for h0 in range(0, H, _KH):
                pltpu.sync_copy(x0_hbm.at[b, c, pl.ds(h0, _KH)], bufx)
                pltpu.sync_copy(noise_hbm.at[b, c, pl.ds(h0, _KH)], bufn)

                def row(r, carry):
                    for l in range(W // 16):
                        s = pl.ds(l * 16, 16)
                        bufo[r, s] = sac * bufx[r, s] + somac * bufn[r, s]
                    return carry

                lax.fori_loop(0, _KH, row, 0)
                pltpu.sync_copy(bufo, out_hbm.at[b, c, pl.ds(h0, _KH)])


def kernel(x0, noise, time_steps, sqrt_alpha_cumprod, sqrt_one_minus_alpha_cumprod):
    B, C, H, W = x0.shape
    ts = time_steps.astype(jnp.int32)
    mesh = plsc.VectorSubcoreMesh(core_axis_name="c", subcore_axis_name="s")

    f = functools.partial(
        pl.kernel,
        mesh=mesh,
        out_type=jax.ShapeDtypeStruct((B, C, H, W), jnp.float32),
        scratch_types=[
            pltpu.VMEM((16,), jnp.int32),
            pltpu.VMEM((16,), jnp.int32),
            pltpu.VMEM((16,), jnp.float32),
            pltpu.VMEM((16,), jnp.float32),
            pltpu.VMEM((_KH, W), jnp.float32),
            pltpu.VMEM((_KH, W), jnp.float32),
            pltpu.VMEM((_KH, W), jnp.float32),
        ],
    )(_sc_body)
    return f(x0, noise, ts, sqrt_alpha_cumprod, sqrt_one_minus_alpha_cumprod)


# speedup vs baseline: 1.9383x; 1.9383x over previous
import functools

import jax
import jax.numpy as jnp
from jax import lax
from jax.experimental import pallas as pl
from jax.experimental.pallas import tpu as pltpu
from jax.experimental.pallas import tpu_sc as plsc

_NC = 2
_NS = 16
_NW = _NC * _NS
_KH = 64
_B_SC = 32
_BLK_B = 8


def _tc_body(ts_ref, sac_ref, somac_ref, x0_ref, noise_ref, out_ref):
    i = pl.program_id(0)
    for j in range(_BLK_B):
        t = ts_ref[i * _BLK_B + j]
        out_ref[j] = sac_ref[t] * x0_ref[j] + somac_ref[t] * noise_ref[j]


def _sc_body(x0_hbm, noise_hbm, ts_hbm, sac_hbm, somac_hbm, out_hbm,
             idx_v, tval_v, sac_v, somac_v, bufx, bufn, bufo):
    B, C, H, W = x0_hbm.shape
    b0 = B - _B_SC
    per_w = _B_SC // _NW
    wid = lax.axis_index("s") * _NC + lax.axis_index("c")

    for k in range(per_w):
        b = b0 + wid * per_w + k
        idx_v[...] = jnp.full((16,), b, dtype=jnp.int32)
        pltpu.sync_copy(ts_hbm.at[idx_v], tval_v)
        pltpu.sync_copy(sac_hbm.at[tval_v], sac_v)
        pltpu.sync_copy(somac_hbm.at[tval_v], somac_v)
        sac = sac_v[...]
        somac = somac_v[...]

        for c in range(C):
            for h0 in range(0, H, _KH):
                pltpu.sync_copy(x0_hbm.at[b, c, pl.ds(h0, _KH)], bufx)
                pltpu.sync_copy(noise_hbm.at[b, c, pl.ds(h0, _KH)], bufn)

                def row(r, carry):
                    for l in range(W // 16):
                        s = pl.ds(l * 16, 16)
                        bufo[r, s] = sac * bufx[r, s] + somac * bufn[r, s]
                    return carry

                lax.fori_loop(0, _KH, row, 0)
                pltpu.sync_copy(
                    bufo, out_hbm.at[b - b0, c, pl.ds(h0, _KH)])


def _stitch(full_ref, sc_ref, out_ref):
    i = pl.program_id(0)
    del full_ref
    out_ref[...] = sc_ref[...]


def kernel(x0, noise, time_steps, sqrt_alpha_cumprod, sqrt_one_minus_alpha_cumprod):
    B, C, H, W = x0.shape
    B1 = B - _B_SC
    ts = time_steps.astype(jnp.int32)

    tc_out = pl.pallas_call(
        _tc_body,
        grid=(B1 // _BLK_B,),
        in_specs=[
            pl.BlockSpec(memory_space=pltpu.SMEM),
            pl.BlockSpec(memory_space=pltpu.SMEM),
            pl.BlockSpec(memory_space=pltpu.SMEM),
            pl.BlockSpec((_BLK_B, C, H, W), lambda b: (b, 0, 0, 0)),
            pl.BlockSpec((_BLK_B, C, H, W), lambda b: (b, 0, 0, 0)),
        ],
        out_specs=pl.BlockSpec((_BLK_B, C, H, W), lambda b: (b, 0, 0, 0)),
        out_shape=jax.ShapeDtypeStruct((B, C, H, W), jnp.float32),
    )(ts, sqrt_alpha_cumprod, sqrt_one_minus_alpha_cumprod, x0, noise)

    sc_mesh = plsc.VectorSubcoreMesh(core_axis_name="c", subcore_axis_name="s")
    sc_part = functools.partial(
        pl.kernel,
        mesh=sc_mesh,
        out_type=jax.ShapeDtypeStruct((_B_SC, C, H, W), jnp.float32),
        scratch_types=[
            pltpu.VMEM((16,), jnp.int32),
            pltpu.VMEM((16,), jnp.int32),
            pltpu.VMEM((16,), jnp.float32),
            pltpu.VMEM((16,), jnp.float32),
            pltpu.VMEM((_KH, W), jnp.float32),
            pltpu.VMEM((_KH, W), jnp.float32),
            pltpu.VMEM((_KH, W), jnp.float32),
        ],
    )(_sc_body)(x0, noise, ts, sqrt_alpha_cumprod, sqrt_one_minus_alpha_cumprod)

    out = pl.pallas_call(
        _stitch,
        grid=(_B_SC // _BLK_B,),
        in_specs=[
            pl.BlockSpec(memory_space=pltpu.HBM),
            pl.BlockSpec((_BLK_B, C, H, W), lambda b: (b, 0, 0, 0)),
        ],
        out_specs=pl.BlockSpec((_BLK_B, C, H, W), lambda b: (b + B1 // _BLK_B, 0, 0, 0)),
        out_shape=jax.ShapeDtypeStruct((B, C, H, W), jnp.float32),
        input_output_aliases={0: 0},
    )(tc_out, sc_part)
    return out


# speedup vs baseline: 2.8260x; 1.4580x over previous
import jax
import jax.numpy as jnp
from jax.experimental import pallas as pl
from jax.experimental.pallas import tpu as pltpu

_BLK_B = 4


def _blend_body(ts_ref, sac_ref, somac_ref, x0_ref, noise_ref, out_ref):
    i = pl.program_id(0)
    for j in range(_BLK_B):
        t = ts_ref[i * _BLK_B + j]
        out_ref[j] = sac_ref[t] * x0_ref[j] + somac_ref[t] * noise_ref[j]


def kernel(x0, noise, time_steps, sqrt_alpha_cumprod, sqrt_one_minus_alpha_cumprod):
    B, C, H, W = x0.shape
    ts = time_steps.astype(jnp.int32)

    out = pl.pallas_call(
        _blend_body,
        grid=(B // _BLK_B,),
        in_specs=[
            pl.BlockSpec(memory_space=pltpu.SMEM),
            pl.BlockSpec(memory_space=pltpu.SMEM),
            pl.BlockSpec(memory_space=pltpu.SMEM),
            pl.BlockSpec((_BLK_B, C, H, W), lambda b: (b, 0, 0, 0)),
            pl.BlockSpec((_BLK_B, C, H, W), lambda b: (b, 0, 0, 0)),
        ],
        out_specs=pl.BlockSpec((_BLK_B, C, H, W), lambda b: (b, 0, 0, 0)),
        out_shape=jax.ShapeDtypeStruct((B, C, H, W), jnp.float32),
    )(ts, sqrt_alpha_cumprod, sqrt_one_minus_alpha_cumprod, x0, noise)
    return out
